# baseline (speedup 1.0000x reference)
import functools

import jax
import jax.numpy as jnp
from jax import lax
from jax.experimental import pallas as pl
from jax.experimental.pallas import tpu as pltpu
from jax.experimental.pallas import tpu_sc as plsc

_B = 1024
_CTX = 50
_EMBED = 16
_VOCAB = 100000

_NC = 2
_NS = 16
_NW = _NC * _NS
_BPW = _B // _NW
_CHUNK_B = 2
_CHUNK = _CHUNK_B * _CTX
_NCHUNK = _BPW // _CHUNK_B
_IDX_PER_W = _BPW * _CTX


def _treesum(vs):
    while len(vs) > 1:
        nxt = [vs[i] + vs[i + 1] for i in range(0, len(vs) - 1, 2)]
        if len(vs) % 2:
            nxt.append(vs[-1])
        vs = nxt
    return vs[0]


def _pool_body(idx_hbm, table_hbm, out_hbm, idx_v, rows_v, pooled_v, sem):
    wid = lax.axis_index("s") * _NC + lax.axis_index("c")
    pltpu.sync_copy(idx_hbm.at[wid], idx_v)
    copies = [
        pltpu.async_copy(
            table_hbm.at[idx_v.at[j]],
            rows_v.at[pl.ds(j * _CHUNK, _CHUNK)],
            sem,
        )
        for j in range(_NCHUNK)
    ]
    for cp in copies:
        cp.wait()

    scale = jnp.full((_EMBED,), 1.0 / _CTX, jnp.float32)

    def body(b, carry):
        base = b * _CTX
        vs = [rows_v[base + j, :] for j in range(_CTX)]
        pooled_v[b, :] = _treesum(vs) * scale
        return carry

    lax.fori_loop(0, _BPW, body, 0)
    pltpu.sync_copy(pooled_v, out_hbm.at[pl.ds(wid * _BPW, _BPW)])


@functools.partial(jax.jit, static_argnames=())
def _pool(idx, table):
    mesh = plsc.VectorSubcoreMesh(core_axis_name="c", subcore_axis_name="s")
    fn = pl.kernel(
        _pool_body,
        out_type=jax.ShapeDtypeStruct((_B, _EMBED), jnp.float32),
        mesh=mesh,
        scratch_types=[
            pltpu.VMEM((_NCHUNK, _CHUNK), jnp.int32),
            pltpu.VMEM((_IDX_PER_W, _EMBED), jnp.float32),
            pltpu.VMEM((_BPW, _EMBED), jnp.float32),
            pltpu.SemaphoreType.DMA,
        ],
        compiler_params=pltpu.CompilerParams(use_tc_tiling_on_sc=False),
    )
    return fn(idx, table)


_VT = 2048
_NVT = (_VOCAB + _VT - 1) // _VT


def _proj_body(x_ref, w_ref, b_ref, o_ref):
    o_ref[...] = (
        jnp.dot(x_ref[...], w_ref[...], preferred_element_type=jnp.float32)
        + b_ref[...]
    )


def _project(x, W, b2d):
    return pl.pallas_call(
        _proj_body,
        grid=(_NVT,),
        in_specs=[
            pl.BlockSpec((_B, _EMBED), lambda i: (0, 0)),
            pl.BlockSpec((_EMBED, _VT), lambda i: (0, i)),
            pl.BlockSpec((1, _VT), lambda i: (0, i)),
        ],
        out_specs=pl.BlockSpec((_B, _VT), lambda i: (0, i)),
        out_shape=jax.ShapeDtypeStruct((_B, _VOCAB), jnp.float32),
    )(x, W, b2d)


def kernel(inputs, emb_table, W, b):
    idx = inputs.astype(jnp.int32).reshape(_NW, _NCHUNK, _CHUNK)
    pooled = _pool(idx, emb_table)
    return _project(pooled, W, b.reshape(1, _VOCAB))
